# Initial kernel scaffold; baseline (speedup 1.0000x reference)
import jax
import jax.numpy as jnp
from jax.experimental import pallas as pl


def kernel(x, edge_index, edge_attr, batch, params):
    raise NotImplementedError("write your pallas kernel here")


# speedup vs baseline: 2.0791x; 2.0791x over previous
import functools

import jax
import jax.numpy as jnp
from jax import lax
from jax.experimental import pallas as pl
from jax.experimental.pallas import tpu as pltpu
from jax.experimental.pallas import tpu_sc as plsc

N = 10000
E = 320000
D = 128
DE = 16
G = 64
DOS_H = 256
DOS_OUT = 400

_NC = 2
_NS = 16
_NW = _NC * _NS
_EW = E // _NW
_B = 80
_NCH = _EW // _B
_NP = 10240
_RPT = _NP // _NS
_ZB = 160

_BN = 2000
_BE = 3200

_f32 = jnp.float32


def _embed_proj_body(x_ref, we_ref, be_ref, wd_ref, ws_ref,
                     h_ref, p_ref, q_ref):
    h = jnp.dot(x_ref[...], we_ref[...], preferred_element_type=_f32)
    h = h + be_ref[...]
    h_ref[...] = h
    p_ref[...] = jnp.dot(h, wd_ref[...], preferred_element_type=_f32)
    q_ref[...] = jnp.dot(h, ws_ref[...], preferred_element_type=_f32)


def _embed_proj(x, we, be, wd, ws):
    grid = (N // _BN,)
    return pl.pallas_call(
        _embed_proj_body,
        grid=grid,
        in_specs=[
            pl.BlockSpec((_BN, D), lambda i: (i, 0)),
            pl.BlockSpec((D, D), lambda i: (0, 0)),
            pl.BlockSpec((1, D), lambda i: (0, 0)),
            pl.BlockSpec((D, 2 * D), lambda i: (0, 0)),
            pl.BlockSpec((D, 2 * D), lambda i: (0, 0)),
        ],
        out_specs=[
            pl.BlockSpec((_BN, D), lambda i: (i, 0)),
            pl.BlockSpec((_BN, 2 * D), lambda i: (i, 0)),
            pl.BlockSpec((_BN, 2 * D), lambda i: (i, 0)),
        ],
        out_shape=[
            jax.ShapeDtypeStruct((N, D), _f32),
            jax.ShapeDtypeStruct((N, 2 * D), _f32),
            jax.ShapeDtypeStruct((N, 2 * D), _f32),
        ],
    )(x, we, be, wd, ws)


def _update_proj_body(h_ref, p0_ref, p1_ref, wd_ref, ws_ref,
                      h_ref_o, p_ref, q_ref):
    h = jax.nn.softplus(h_ref[...] + p0_ref[...] + p1_ref[...])
    h_ref_o[...] = h
    p_ref[...] = jnp.dot(h, wd_ref[...], preferred_element_type=_f32)
    q_ref[...] = jnp.dot(h, ws_ref[...], preferred_element_type=_f32)


def _update_proj(h, p0, p1, wd, ws):
    grid = (N // _BN,)
    return pl.pallas_call(
        _update_proj_body,
        grid=grid,
        in_specs=[
            pl.BlockSpec((_BN, D), lambda i: (i, 0)),
            pl.BlockSpec((_BN, D), lambda i: (i, 0)),
            pl.BlockSpec((_BN, D), lambda i: (i, 0)),
            pl.BlockSpec((D, 2 * D), lambda i: (0, 0)),
            pl.BlockSpec((D, 2 * D), lambda i: (0, 0)),
        ],
        out_specs=[
            pl.BlockSpec((_BN, D), lambda i: (i, 0)),
            pl.BlockSpec((_BN, 2 * D), lambda i: (i, 0)),
            pl.BlockSpec((_BN, 2 * D), lambda i: (i, 0)),
        ],
        out_shape=[
            jax.ShapeDtypeStruct((N, D), _f32),
            jax.ShapeDtypeStruct((N, 2 * D), _f32),
            jax.ShapeDtypeStruct((N, 2 * D), _f32),
        ],
    )(h, p0, p1, wd, ws)


def _msg_body(z_ref, ea_ref, wge_ref, wce_ref, bg_ref, bc_ref, msg_ref):
    z = z_ref[...]
    ea = ea_ref[...]
    g = z[:, :D] + jnp.dot(ea, wge_ref[...], preferred_element_type=_f32)
    g = g + bg_ref[...]
    c = z[:, D:] + jnp.dot(ea, wce_ref[...], preferred_element_type=_f32)
    c = c + bc_ref[...]
    msg_ref[...] = jax.nn.sigmoid(g) * jax.nn.softplus(c)


def _msg(zsum, ea, wge, wce, bg, bc):
    grid = (E // _BE,)
    return pl.pallas_call(
        _msg_body,
        grid=grid,
        in_specs=[
            pl.BlockSpec((_BE, 2 * D), lambda i: (i, 0)),
            pl.BlockSpec((_BE, DE), lambda i: (i, 0)),
            pl.BlockSpec((DE, D), lambda i: (0, 0)),
            pl.BlockSpec((DE, D), lambda i: (0, 0)),
            pl.BlockSpec((1, D), lambda i: (0, 0)),
            pl.BlockSpec((1, D), lambda i: (0, 0)),
        ],
        out_specs=pl.BlockSpec((_BE, D), lambda i: (i, 0)),
        out_shape=jax.ShapeDtypeStruct((E, D), _f32),
    )(zsum, ea, wge, wce, bg, bc)


def _finalh_body(h_ref, p0_ref, p1_ref, wsite_ref, bsite_ref,
                 h_o_ref, site_ref):
    h = jax.nn.softplus(h_ref[...] + p0_ref[...] + p1_ref[...])
    h_o_ref[...] = h
    site_ref[...] = (jnp.dot(h, wsite_ref[...], preferred_element_type=_f32)
                     + bsite_ref[...])


def _finalh(h, p0, p1, wsite, bsite):
    grid = (N // _BN,)
    return pl.pallas_call(
        _finalh_body,
        grid=grid,
        in_specs=[
            pl.BlockSpec((_BN, D), lambda i: (i, 0)),
            pl.BlockSpec((_BN, D), lambda i: (i, 0)),
            pl.BlockSpec((_BN, D), lambda i: (i, 0)),
            pl.BlockSpec((D, 1), lambda i: (0, 0)),
            pl.BlockSpec((1, 1), lambda i: (0, 0)),
        ],
        out_specs=[
            pl.BlockSpec((_BN, D), lambda i: (i, 0)),
            pl.BlockSpec((_BN, 1), lambda i: (i, 0)),
        ],
        out_shape=[
            jax.ShapeDtypeStruct((N, D), _f32),
            jax.ShapeDtypeStruct((N, 1), _f32),
        ],
    )(h, p0, p1, wsite, bsite)


def _heads_body(h_ref, b_ref, wf_ref, bf_ref, wd1_ref, bd1_ref,
                wd2_ref, bd2_ref, wde_ref, bde_ref, wdg_ref, bdg_ref,
                form_ref, dos_ref, desc_ref, dgh_ref):
    b = b_ref[...]
    gid = lax.broadcasted_iota(jnp.int32, (G, 1), 0)
    maskt = (b == gid).astype(_f32)
    counts = jnp.sum(maskt, axis=1, keepdims=True)
    csum = jnp.dot(maskt, h_ref[...], preferred_element_type=_f32)
    crystal = csum / jnp.maximum(counts, 1.0)
    form_ref[...] = (jnp.dot(crystal, wf_ref[...], preferred_element_type=_f32)
                     + bf_ref[...])
    d1 = jax.nn.relu(jnp.dot(crystal, wd1_ref[...], preferred_element_type=_f32)
                     + bd1_ref[...])
    dos_ref[...] = (jnp.dot(d1, wd2_ref[...], preferred_element_type=_f32)
                    + bd2_ref[...])
    desc_ref[...] = (jnp.dot(crystal, wde_ref[...], preferred_element_type=_f32)
                     + bde_ref[...])
    dgh_ref[...] = (jnp.dot(crystal, wdg_ref[...], preferred_element_type=_f32)
                    + bdg_ref[...])


def _heads(h3, batch2d, wf, bf, wd1, bd1, wd2, bd2, wde, bde, wdg, bdg):
    return pl.pallas_call(
        _heads_body,
        out_shape=[
            jax.ShapeDtypeStruct((G, 1), _f32),
            jax.ShapeDtypeStruct((G, DOS_OUT), _f32),
            jax.ShapeDtypeStruct((G, DESC), _f32),
            jax.ShapeDtypeStruct((G, 1), _f32),
        ],
    )(h3, batch2d, wf, bf, wd1, bd1, wd2, bd2, wde, bde, wdg, bdg)


DESC = 8


def _sc_gather_body(p_hbm, q_hbm, dst_hbm, src_hbm, z_hbm,
                    idxd, idxs, pd, qs, semd, sems):
    c = lax.axis_index("c")
    s = lax.axis_index("s")
    wid = s * _NC + c
    base0 = wid * _EW

    def chunk(j, carry):
        base = base0 + j * _B
        pltpu.sync_copy(dst_hbm.at[pl.ds(base, _B)], idxd)
        pltpu.sync_copy(src_hbm.at[pl.ds(base, _B)], idxs)
        cp1 = pltpu.async_copy(p_hbm.at[idxd], pd, semd)
        cp2 = pltpu.async_copy(q_hbm.at[idxs], qs, sems)
        cp1.wait()
        cp2.wait()

        def row(i, cc):
            for k in range(16):
                sl = pl.ds(k * 16, 16)
                qs[i, sl] = qs[i, sl] + pd[i, sl]
            return cc

        lax.fori_loop(0, _B, row, 0)
        pltpu.sync_copy(qs, z_hbm.at[pl.ds(base, _B)])
        return carry

    lax.fori_loop(0, _NCH, chunk, 0)


@functools.lru_cache(maxsize=None)
def _make_sc_gather():
    mesh = plsc.VectorSubcoreMesh(core_axis_name="c", subcore_axis_name="s",
                                  num_cores=_NC, num_subcores=_NS)
    return pl.kernel(
        _sc_gather_body,
        out_type=jax.ShapeDtypeStruct((E, 2 * D), _f32),
        mesh=mesh,
        scratch_types=[
            pltpu.VMEM((_B,), jnp.int32),
            pltpu.VMEM((_B,), jnp.int32),
            pltpu.VMEM((_B, 2 * D), _f32),
            pltpu.VMEM((_B, 2 * D), _f32),
            pltpu.SemaphoreType.DMA,
            pltpu.SemaphoreType.DMA,
        ],
    )


def _sc_gather(p, q, dst, src):
    return _make_sc_gather()(p, q, dst, src)


def _sc_scatter_body(msg_hbm, dst_hbm, out_hbm, acc, zbuf, idxd, mv, _sem):
    c = lax.axis_index("c")
    s = lax.axis_index("s")

    def zrow(i, cc):
        for k in range(D // 16):
            zbuf[i, pl.ds(k * 16, 16)] = jnp.zeros((16,), _f32)
        return cc

    lax.fori_loop(0, _ZB, zrow, 0)
    for j in range(_RPT // _ZB):
        pltpu.sync_copy(zbuf, acc.at[pl.ds(s * _RPT + j * _ZB, _ZB)])
    plsc.subcore_barrier()

    base0 = (c * _NS + s) * _EW

    def chunk(j, cc):
        base = base0 + j * _B
        pltpu.sync_copy(dst_hbm.at[pl.ds(base, _B)], idxd)
        pltpu.sync_copy(msg_hbm.at[pl.ds(base, _B)], mv)
        pltpu.sync_copy(mv, acc.at[idxd], add=True)
        return cc

    lax.fori_loop(0, _NCH, chunk, 0)
    plsc.subcore_barrier()

    for j in range(_RPT // _ZB):
        r0 = s * _RPT + j * _ZB
        pltpu.sync_copy(acc.at[pl.ds(r0, _ZB)], out_hbm.at[c, pl.ds(r0, _ZB)])


@functools.lru_cache(maxsize=None)
def _make_sc_scatter():
    mesh = plsc.VectorSubcoreMesh(core_axis_name="c", subcore_axis_name="s",
                                  num_cores=_NC, num_subcores=_NS)
    return pl.kernel(
        _sc_scatter_body,
        out_type=jax.ShapeDtypeStruct((_NC, _NP, D), _f32),
        mesh=mesh,
        scratch_types=[
            pltpu.VMEM_SHARED((_NP, D), _f32),
            pltpu.VMEM((_ZB, D), _f32),
            pltpu.VMEM((_B,), jnp.int32),
            pltpu.VMEM((_B, D), _f32),
            pltpu.SemaphoreType.DMA,
        ],
    )


def _sc_scatter(msg, dst):
    return _make_sc_scatter()(msg, dst)[:, :N, :]


def kernel(x, edge_index, edge_attr, batch, params):
    src = edge_index[0]
    dst = edge_index[1]
    convs = params['convs']

    wd = [jnp.concatenate([cv['Wg'][:D], cv['Wc'][:D]], axis=1) for cv in convs]
    ws = [jnp.concatenate([cv['Wg'][D:2 * D], cv['Wc'][D:2 * D]], axis=1)
          for cv in convs]
    wge = [cv['Wg'][2 * D:] for cv in convs]
    wce = [cv['Wc'][2 * D:] for cv in convs]
    bg = [cv['bg'].reshape(1, D) for cv in convs]
    bc = [cv['bc'].reshape(1, D) for cv in convs]

    h, p, q = _embed_proj(x, params['W_embed'],
                          params['b_embed'].reshape(1, D), wd[0], ws[0])

    for l in range(3):
        zsum = _sc_gather(p, q, dst, src)
        msg = _msg(zsum, edge_attr, wge[l], wce[l], bg[l], bc[l])
        parts = _sc_scatter(msg, dst)
        if l < 2:
            h, p, q = _update_proj(h, parts[0], parts[1], wd[l + 1], ws[l + 1])
        else:
            h3, site = _finalh(h, parts[0], parts[1], params['W_site'],
                               params['b_site'].reshape(1, 1))

    formation, dos, desc, dgh = _heads(
        h3, batch.reshape(1, N),
        params['W_form'], params['b_form'].reshape(1, 1),
        params['W_dos1'], params['b_dos1'].reshape(1, DOS_H),
        params['W_dos2'], params['b_dos2'].reshape(1, DOS_OUT),
        params['W_desc'], params['b_desc'].reshape(1, DESC),
        params['W_dgh'], params['b_dgh'].reshape(1, 1),
    )
    return (formation, site, dos, desc, dgh)


# speedup vs baseline: 3.1475x; 1.5139x over previous
import functools

import jax
import jax.numpy as jnp
from jax import lax
from jax.experimental import pallas as pl
from jax.experimental.pallas import tpu as pltpu
from jax.experimental.pallas import tpu_sc as plsc

N = 10000
E = 320000
D = 128
DE = 16
G = 64
DOS_H = 256
DOS_OUT = 400

_NC = 2
_NS = 16
_NW = _NC * _NS
_EW = E // _NW
_B = 40
_NCH = _EW // _B
_NP = 10240
_RPT = _NP // _NS
_ZB = 160

_BN = 2000
_BE = 3200

_f32 = jnp.float32


def _embed_proj_body(x_ref, we_ref, be_ref, wd_ref, ws_ref,
                     h_ref, p_ref, q_ref):
    h = jnp.dot(x_ref[...], we_ref[...], preferred_element_type=_f32)
    h = h + be_ref[...]
    h_ref[...] = h
    p_ref[...] = jnp.dot(h, wd_ref[...], preferred_element_type=_f32)
    q_ref[...] = jnp.dot(h, ws_ref[...], preferred_element_type=_f32)


def _embed_proj(x, we, be, wd, ws):
    grid = (N // _BN,)
    return pl.pallas_call(
        _embed_proj_body,
        grid=grid,
        in_specs=[
            pl.BlockSpec((_BN, D), lambda i: (i, 0)),
            pl.BlockSpec((D, D), lambda i: (0, 0)),
            pl.BlockSpec((1, D), lambda i: (0, 0)),
            pl.BlockSpec((D, 2 * D), lambda i: (0, 0)),
            pl.BlockSpec((D, 2 * D), lambda i: (0, 0)),
        ],
        out_specs=[
            pl.BlockSpec((_BN, D), lambda i: (i, 0)),
            pl.BlockSpec((_BN, 2 * D), lambda i: (i, 0)),
            pl.BlockSpec((_BN, 2 * D), lambda i: (i, 0)),
        ],
        out_shape=[
            jax.ShapeDtypeStruct((N, D), _f32),
            jax.ShapeDtypeStruct((N, 2 * D), _f32),
            jax.ShapeDtypeStruct((N, 2 * D), _f32),
        ],
    )(x, we, be, wd, ws)


def _update_proj_body(h_ref, p0_ref, p1_ref, wd_ref, ws_ref,
                      h_ref_o, p_ref, q_ref):
    h = jax.nn.softplus(h_ref[...] + p0_ref[...] + p1_ref[...])
    h_ref_o[...] = h
    p_ref[...] = jnp.dot(h, wd_ref[...], preferred_element_type=_f32)
    q_ref[...] = jnp.dot(h, ws_ref[...], preferred_element_type=_f32)


def _update_proj(h, p0, p1, wd, ws):
    grid = (N // _BN,)
    return pl.pallas_call(
        _update_proj_body,
        grid=grid,
        in_specs=[
            pl.BlockSpec((_BN, D), lambda i: (i, 0)),
            pl.BlockSpec((_BN, D), lambda i: (i, 0)),
            pl.BlockSpec((_BN, D), lambda i: (i, 0)),
            pl.BlockSpec((D, 2 * D), lambda i: (0, 0)),
            pl.BlockSpec((D, 2 * D), lambda i: (0, 0)),
        ],
        out_specs=[
            pl.BlockSpec((_BN, D), lambda i: (i, 0)),
            pl.BlockSpec((_BN, 2 * D), lambda i: (i, 0)),
            pl.BlockSpec((_BN, 2 * D), lambda i: (i, 0)),
        ],
        out_shape=[
            jax.ShapeDtypeStruct((N, D), _f32),
            jax.ShapeDtypeStruct((N, 2 * D), _f32),
            jax.ShapeDtypeStruct((N, 2 * D), _f32),
        ],
    )(h, p0, p1, wd, ws)


def _msg_body(z_ref, ea_ref, wge_ref, wce_ref, bg_ref, bc_ref, msg_ref):
    z = z_ref[...]
    ea = ea_ref[...]
    g = z[:, :D] + jnp.dot(ea, wge_ref[...], preferred_element_type=_f32)
    g = g + bg_ref[...]
    c = z[:, D:] + jnp.dot(ea, wce_ref[...], preferred_element_type=_f32)
    c = c + bc_ref[...]
    msg_ref[...] = jax.nn.sigmoid(g) * jax.nn.softplus(c)


def _msg(zsum, ea, wge, wce, bg, bc):
    grid = (E // _BE,)
    return pl.pallas_call(
        _msg_body,
        grid=grid,
        in_specs=[
            pl.BlockSpec((_BE, 2 * D), lambda i: (i, 0)),
            pl.BlockSpec((_BE, DE), lambda i: (i, 0)),
            pl.BlockSpec((DE, D), lambda i: (0, 0)),
            pl.BlockSpec((DE, D), lambda i: (0, 0)),
            pl.BlockSpec((1, D), lambda i: (0, 0)),
            pl.BlockSpec((1, D), lambda i: (0, 0)),
        ],
        out_specs=pl.BlockSpec((_BE, D), lambda i: (i, 0)),
        out_shape=jax.ShapeDtypeStruct((E, D), _f32),
    )(zsum, ea, wge, wce, bg, bc)


def _finalh_body(h_ref, p0_ref, p1_ref, wsite_ref, bsite_ref,
                 h_o_ref, site_ref):
    h = jax.nn.softplus(h_ref[...] + p0_ref[...] + p1_ref[...])
    h_o_ref[...] = h
    site_ref[...] = (jnp.dot(h, wsite_ref[...], preferred_element_type=_f32)
                     + bsite_ref[...])


def _finalh(h, p0, p1, wsite, bsite):
    grid = (N // _BN,)
    return pl.pallas_call(
        _finalh_body,
        grid=grid,
        in_specs=[
            pl.BlockSpec((_BN, D), lambda i: (i, 0)),
            pl.BlockSpec((_BN, D), lambda i: (i, 0)),
            pl.BlockSpec((_BN, D), lambda i: (i, 0)),
            pl.BlockSpec((D, 1), lambda i: (0, 0)),
            pl.BlockSpec((1, 1), lambda i: (0, 0)),
        ],
        out_specs=[
            pl.BlockSpec((_BN, D), lambda i: (i, 0)),
            pl.BlockSpec((_BN, 1), lambda i: (i, 0)),
        ],
        out_shape=[
            jax.ShapeDtypeStruct((N, D), _f32),
            jax.ShapeDtypeStruct((N, 1), _f32),
        ],
    )(h, p0, p1, wsite, bsite)


def _heads_body(h_ref, b_ref, wf_ref, bf_ref, wd1_ref, bd1_ref,
                wd2_ref, bd2_ref, wde_ref, bde_ref, wdg_ref, bdg_ref,
                form_ref, dos_ref, desc_ref, dgh_ref):
    b = b_ref[...]
    gid = lax.broadcasted_iota(jnp.int32, (G, 1), 0)
    maskt = (b == gid).astype(_f32)
    counts = jnp.sum(maskt, axis=1, keepdims=True)
    csum = jnp.dot(maskt, h_ref[...], preferred_element_type=_f32)
    crystal = csum / jnp.maximum(counts, 1.0)
    form_ref[...] = (jnp.dot(crystal, wf_ref[...], preferred_element_type=_f32)
                     + bf_ref[...])
    d1 = jax.nn.relu(jnp.dot(crystal, wd1_ref[...], preferred_element_type=_f32)
                     + bd1_ref[...])
    dos_ref[...] = (jnp.dot(d1, wd2_ref[...], preferred_element_type=_f32)
                    + bd2_ref[...])
    desc_ref[...] = (jnp.dot(crystal, wde_ref[...], preferred_element_type=_f32)
                     + bde_ref[...])
    dgh_ref[...] = (jnp.dot(crystal, wdg_ref[...], preferred_element_type=_f32)
                    + bdg_ref[...])


def _heads(h3, batch2d, wf, bf, wd1, bd1, wd2, bd2, wde, bde, wdg, bdg):
    return pl.pallas_call(
        _heads_body,
        out_shape=[
            jax.ShapeDtypeStruct((G, 1), _f32),
            jax.ShapeDtypeStruct((G, DOS_OUT), _f32),
            jax.ShapeDtypeStruct((G, DESC), _f32),
            jax.ShapeDtypeStruct((G, 1), _f32),
        ],
    )(h3, batch2d, wf, bf, wd1, bd1, wd2, bd2, wde, bde, wdg, bdg)


DESC = 8


def _sc_gather_body(p_hbm, q_hbm, dst_hbm, src_hbm, z_hbm,
                    idxd, idxs, pd, qs, st,
                    semi0, semi1, semg0, semg1, semst0, semst1):
    c = lax.axis_index("c")
    s = lax.axis_index("s")
    wid = s * _NC + c
    base0 = wid * _EW
    semi = (semi0, semi1)
    semg = (semg0, semg1)
    semst = (semst0, semst1)

    def start_idx(j, b):
        base = base0 + j * _B
        pltpu.async_copy(dst_hbm.at[pl.ds(base, _B)], idxd.at[b], semi[b])
        pltpu.async_copy(src_hbm.at[pl.ds(base, _B)], idxs.at[b], semi[b])

    def wait_idx(b):
        pltpu.make_async_copy(dst_hbm.at[pl.ds(0, _B)], idxd.at[b],
                              semi[b]).wait()
        pltpu.make_async_copy(src_hbm.at[pl.ds(0, _B)], idxs.at[b],
                              semi[b]).wait()

    def start_gather(b):
        pltpu.async_copy(p_hbm.at[idxd.at[b]], pd.at[b], semg[b])
        pltpu.async_copy(q_hbm.at[idxs.at[b]], qs.at[b], semg[b])

    def wait_gather(b):
        pltpu.make_async_copy(p_hbm.at[idxd.at[b]], pd.at[b], semg[b]).wait()
        pltpu.make_async_copy(q_hbm.at[idxs.at[b]], qs.at[b], semg[b]).wait()

    def start_store(j, b):
        base = base0 + j * _B
        pltpu.async_copy(st.at[b], z_hbm.at[pl.ds(base, _B)], semst[b])

    def wait_store(b):
        pltpu.make_async_copy(st.at[b], z_hbm.at[pl.ds(0, _B)],
                              semst[b]).wait()

    start_idx(0, 0)
    wait_idx(0)
    start_gather(0)
    start_idx(1, 1)

    def sub(j, b):
        b1 = 1 - b

        @pl.when(j + 1 < _NCH)
        def _():
            wait_idx(b1)
            start_gather(b1)

        wait_gather(b)

        @pl.when(j + 2 < _NCH)
        def _():
            start_idx(j + 2, b)

        @pl.when(j >= 2)
        def _():
            wait_store(b)

        pdb, qsb, stb = pd.at[b], qs.at[b], st.at[b]

        def row(i, cc):
            for k in range(2 * D // 16):
                sl = pl.ds(k * 16, 16)
                stb[i, sl] = pdb[i, sl] + qsb[i, sl]
            return cc

        lax.fori_loop(0, _B, row, 0)
        start_store(j, b)

    def body(i, cc):
        sub(2 * i, 0)
        sub(2 * i + 1, 1)
        return cc

    lax.fori_loop(0, _NCH // 2, body, 0)
    wait_store(0)
    wait_store(1)


@functools.lru_cache(maxsize=None)
def _make_sc_gather():
    mesh = plsc.VectorSubcoreMesh(core_axis_name="c", subcore_axis_name="s",
                                  num_cores=_NC, num_subcores=_NS)
    return pl.kernel(
        _sc_gather_body,
        out_type=jax.ShapeDtypeStruct((E, 2 * D), _f32),
        mesh=mesh,
        scratch_types=[
            pltpu.VMEM((2, _B), jnp.int32),
            pltpu.VMEM((2, _B), jnp.int32),
            pltpu.VMEM((2, _B, 2 * D), _f32),
            pltpu.VMEM((2, _B, 2 * D), _f32),
            pltpu.VMEM((2, _B, 2 * D), _f32),
            pltpu.SemaphoreType.DMA,
            pltpu.SemaphoreType.DMA,
            pltpu.SemaphoreType.DMA,
            pltpu.SemaphoreType.DMA,
            pltpu.SemaphoreType.DMA,
            pltpu.SemaphoreType.DMA,
        ],
    )


def _sc_gather(p, q, dst, src):
    return _make_sc_gather()(p, q, dst, src)


def _sc_scatter_body(msg_hbm, dst_hbm, out_hbm, acc, zbuf, idx, mv,
                     seml0, seml1, sems0, sems1):
    c = lax.axis_index("c")
    s = lax.axis_index("s")
    seml = (seml0, seml1)
    sems = (sems0, sems1)

    def zrow(i, cc):
        for k in range(D // 16):
            zbuf[i, pl.ds(k * 16, 16)] = jnp.zeros((16,), _f32)
        return cc

    lax.fori_loop(0, _ZB, zrow, 0)
    for j in range(_RPT // _ZB):
        pltpu.sync_copy(zbuf, acc.at[pl.ds(s * _RPT + j * _ZB, _ZB)])
    plsc.subcore_barrier()

    base0 = (c * _NS + s) * _EW

    def start_load(j, b):
        base = base0 + j * _B
        pltpu.async_copy(dst_hbm.at[pl.ds(base, _B)], idx.at[b], seml[b])
        pltpu.async_copy(msg_hbm.at[pl.ds(base, _B)], mv.at[b], seml[b])

    def wait_load(b):
        pltpu.make_async_copy(dst_hbm.at[pl.ds(0, _B)], idx.at[b],
                              seml[b]).wait()
        pltpu.make_async_copy(msg_hbm.at[pl.ds(0, _B)], mv.at[b],
                              seml[b]).wait()

    def start_scat(b):
        pltpu.async_copy(mv.at[b], acc.at[idx.at[b]], sems[b], add=True)

    def wait_scat(b):
        pltpu.make_async_copy(mv.at[b], acc.at[idx.at[b]], sems[b]).wait()

    start_load(0, 0)

    def sub(j, b):
        b1 = 1 - b

        @pl.when(j + 1 < _NCH)
        def _():
            @pl.when(j >= 1)
            def _():
                wait_scat(b1)
            start_load(j + 1, b1)

        wait_load(b)
        start_scat(b)

    def body(i, cc):
        sub(2 * i, 0)
        sub(2 * i + 1, 1)
        return cc

    lax.fori_loop(0, _NCH // 2, body, 0)
    wait_scat(0)
    wait_scat(1)
    plsc.subcore_barrier()

    for j in range(_RPT // _ZB):
        r0 = s * _RPT + j * _ZB
        pltpu.sync_copy(acc.at[pl.ds(r0, _ZB)], out_hbm.at[c, pl.ds(r0, _ZB)])


@functools.lru_cache(maxsize=None)
def _make_sc_scatter():
    mesh = plsc.VectorSubcoreMesh(core_axis_name="c", subcore_axis_name="s",
                                  num_cores=_NC, num_subcores=_NS)
    return pl.kernel(
        _sc_scatter_body,
        out_type=jax.ShapeDtypeStruct((_NC, _NP, D), _f32),
        mesh=mesh,
        scratch_types=[
            pltpu.VMEM_SHARED((_NP, D), _f32),
            pltpu.VMEM((_ZB, D), _f32),
            pltpu.VMEM((2, _B), jnp.int32),
            pltpu.VMEM((2, _B, D), _f32),
            pltpu.SemaphoreType.DMA,
            pltpu.SemaphoreType.DMA,
            pltpu.SemaphoreType.DMA,
            pltpu.SemaphoreType.DMA,
        ],
    )


def _sc_scatter(msg, dst):
    return _make_sc_scatter()(msg, dst)[:, :N, :]


def kernel(x, edge_index, edge_attr, batch, params):
    src = edge_index[0]
    dst = edge_index[1]
    convs = params['convs']

    wd = [jnp.concatenate([cv['Wg'][:D], cv['Wc'][:D]], axis=1) for cv in convs]
    ws = [jnp.concatenate([cv['Wg'][D:2 * D], cv['Wc'][D:2 * D]], axis=1)
          for cv in convs]
    wge = [cv['Wg'][2 * D:] for cv in convs]
    wce = [cv['Wc'][2 * D:] for cv in convs]
    bg = [cv['bg'].reshape(1, D) for cv in convs]
    bc = [cv['bc'].reshape(1, D) for cv in convs]

    h, p, q = _embed_proj(x, params['W_embed'],
                          params['b_embed'].reshape(1, D), wd[0], ws[0])

    for l in range(3):
        zsum = _sc_gather(p, q, dst, src)
        msg = _msg(zsum, edge_attr, wge[l], wce[l], bg[l], bc[l])
        parts = _sc_scatter(msg, dst)
        if l < 2:
            h, p, q = _update_proj(h, parts[0], parts[1], wd[l + 1], ws[l + 1])
        else:
            h3, site = _finalh(h, parts[0], parts[1], params['W_site'],
                               params['b_site'].reshape(1, 1))

    formation, dos, desc, dgh = _heads(
        h3, batch.reshape(1, N),
        params['W_form'], params['b_form'].reshape(1, 1),
        params['W_dos1'], params['b_dos1'].reshape(1, DOS_H),
        params['W_dos2'], params['b_dos2'].reshape(1, DOS_OUT),
        params['W_desc'], params['b_desc'].reshape(1, DESC),
        params['W_dgh'], params['b_dgh'].reshape(1, 1),
    )
    return (formation, site, dos, desc, dgh)


# speedup vs baseline: 3.5176x; 1.1176x over previous
import functools

import jax
import jax.numpy as jnp
from jax import lax
from jax.experimental import pallas as pl
from jax.experimental.pallas import tpu as pltpu
from jax.experimental.pallas import tpu_sc as plsc

N = 10000
E = 320000
D = 128
DE = 16
G = 64
DOS_H = 256
DOS_OUT = 400

_NC = 2
_NS = 16
_NW = _NC * _NS
_EW = E // _NW
_B = 40
_NCH = _EW // _B
_NP = 10240
_RPT = _NP // _NS
_ZB = 160

_BN = 2000
_BE = 3200

_f32 = jnp.float32


def _embed_proj_body(x_ref, we_ref, be_ref, wd_ref, ws_ref,
                     h_ref, p_ref, q_ref):
    h = jnp.dot(x_ref[...], we_ref[...], preferred_element_type=_f32)
    h = h + be_ref[...]
    h_ref[...] = h
    p_ref[...] = jnp.dot(h, wd_ref[...], preferred_element_type=_f32)
    q_ref[...] = jnp.dot(h, ws_ref[...], preferred_element_type=_f32)


def _embed_proj(x, we, be, wd, ws):
    grid = (N // _BN,)
    return pl.pallas_call(
        _embed_proj_body,
        grid=grid,
        in_specs=[
            pl.BlockSpec((_BN, D), lambda i: (i, 0)),
            pl.BlockSpec((D, D), lambda i: (0, 0)),
            pl.BlockSpec((1, D), lambda i: (0, 0)),
            pl.BlockSpec((D, 2 * D), lambda i: (0, 0)),
            pl.BlockSpec((D, 2 * D), lambda i: (0, 0)),
        ],
        out_specs=[
            pl.BlockSpec((_BN, D), lambda i: (i, 0)),
            pl.BlockSpec((_BN, 2 * D), lambda i: (i, 0)),
            pl.BlockSpec((_BN, 2 * D), lambda i: (i, 0)),
        ],
        out_shape=[
            jax.ShapeDtypeStruct((N, D), _f32),
            jax.ShapeDtypeStruct((N, 2 * D), _f32),
            jax.ShapeDtypeStruct((N, 2 * D), _f32),
        ],
    )(x, we, be, wd, ws)


def _update_proj_body(h_ref, pa_ref, pb_ref, wd_ref, ws_ref,
                      h_ref_o, p_ref, q_ref):
    agg = pa_ref[0] + pa_ref[1] + pb_ref[0] + pb_ref[1]
    h = jax.nn.softplus(h_ref[...] + agg)
    h_ref_o[...] = h
    p_ref[...] = jnp.dot(h, wd_ref[...], preferred_element_type=_f32)
    q_ref[...] = jnp.dot(h, ws_ref[...], preferred_element_type=_f32)


def _update_proj(h, pa, pb, wd, ws):
    grid = (N // _BN,)
    return pl.pallas_call(
        _update_proj_body,
        grid=grid,
        in_specs=[
            pl.BlockSpec((_BN, D), lambda i: (i, 0)),
            pl.BlockSpec((_NC, _BN, D), lambda i: (0, i, 0)),
            pl.BlockSpec((_NC, _BN, D), lambda i: (0, i, 0)),
            pl.BlockSpec((D, 2 * D), lambda i: (0, 0)),
            pl.BlockSpec((D, 2 * D), lambda i: (0, 0)),
        ],
        out_specs=[
            pl.BlockSpec((_BN, D), lambda i: (i, 0)),
            pl.BlockSpec((_BN, 2 * D), lambda i: (i, 0)),
            pl.BlockSpec((_BN, 2 * D), lambda i: (i, 0)),
        ],
        out_shape=[
            jax.ShapeDtypeStruct((N, D), _f32),
            jax.ShapeDtypeStruct((N, 2 * D), _f32),
            jax.ShapeDtypeStruct((N, 2 * D), _f32),
        ],
    )(h, pa, pb, wd, ws)


def _msg_body(z_ref, ea_ref, wge_ref, wce_ref, bg_ref, bc_ref, msg_ref):
    z = z_ref[...]
    ea = ea_ref[...]
    g = z[:, :D] + jnp.dot(ea, wge_ref[...], preferred_element_type=_f32)
    g = g + bg_ref[...]
    c = z[:, D:] + jnp.dot(ea, wce_ref[...], preferred_element_type=_f32)
    c = c + bc_ref[...]
    msg_ref[...] = jax.nn.sigmoid(g) * jax.nn.softplus(c)


def _msg(zsum, ea, wge, wce, bg, bc, half):
    ne = zsum.shape[0]
    nblk = ne // _BE
    off = half * nblk
    return pl.pallas_call(
        _msg_body,
        grid=(nblk,),
        in_specs=[
            pl.BlockSpec((_BE, 2 * D), lambda i: (i, 0)),
            pl.BlockSpec((_BE, DE), lambda i: (i + off, 0)),
            pl.BlockSpec((DE, D), lambda i: (0, 0)),
            pl.BlockSpec((DE, D), lambda i: (0, 0)),
            pl.BlockSpec((1, D), lambda i: (0, 0)),
            pl.BlockSpec((1, D), lambda i: (0, 0)),
        ],
        out_specs=pl.BlockSpec((_BE, D), lambda i: (i, 0)),
        out_shape=jax.ShapeDtypeStruct((ne, D), _f32),
    )(zsum, ea, wge, wce, bg, bc)


def _finalh_body(h_ref, pa_ref, pb_ref, wsite_ref, bsite_ref,
                 h_o_ref, site_ref):
    agg = pa_ref[0] + pa_ref[1] + pb_ref[0] + pb_ref[1]
    h = jax.nn.softplus(h_ref[...] + agg)
    h_o_ref[...] = h
    site_ref[...] = (jnp.dot(h, wsite_ref[...], preferred_element_type=_f32)
                     + bsite_ref[...])


def _finalh(h, pa, pb, wsite, bsite):
    grid = (N // _BN,)
    return pl.pallas_call(
        _finalh_body,
        grid=grid,
        in_specs=[
            pl.BlockSpec((_BN, D), lambda i: (i, 0)),
            pl.BlockSpec((_NC, _BN, D), lambda i: (0, i, 0)),
            pl.BlockSpec((_NC, _BN, D), lambda i: (0, i, 0)),
            pl.BlockSpec((D, 1), lambda i: (0, 0)),
            pl.BlockSpec((1, 1), lambda i: (0, 0)),
        ],
        out_specs=[
            pl.BlockSpec((_BN, D), lambda i: (i, 0)),
            pl.BlockSpec((_BN, 1), lambda i: (i, 0)),
        ],
        out_shape=[
            jax.ShapeDtypeStruct((N, D), _f32),
            jax.ShapeDtypeStruct((N, 1), _f32),
        ],
    )(h, pa, pb, wsite, bsite)


def _heads_body(h_ref, b_ref, wf_ref, bf_ref, wd1_ref, bd1_ref,
                wd2_ref, bd2_ref, wde_ref, bde_ref, wdg_ref, bdg_ref,
                form_ref, dos_ref, desc_ref, dgh_ref):
    b = b_ref[...]
    gid = lax.broadcasted_iota(jnp.int32, (G, 1), 0)
    maskt = (b == gid).astype(_f32)
    counts = jnp.sum(maskt, axis=1, keepdims=True)
    csum = jnp.dot(maskt, h_ref[...], preferred_element_type=_f32,
                   precision=lax.Precision.HIGHEST)
    crystal = csum / jnp.maximum(counts, 1.0)
    form_ref[...] = (jnp.dot(crystal, wf_ref[...], preferred_element_type=_f32)
                     + bf_ref[...])
    d1 = jax.nn.relu(jnp.dot(crystal, wd1_ref[...], preferred_element_type=_f32)
                     + bd1_ref[...])
    dos_ref[...] = (jnp.dot(d1, wd2_ref[...], preferred_element_type=_f32)
                    + bd2_ref[...])
    desc_ref[...] = (jnp.dot(crystal, wde_ref[...], preferred_element_type=_f32)
                     + bde_ref[...])
    dgh_ref[...] = (jnp.dot(crystal, wdg_ref[...], preferred_element_type=_f32)
                    + bdg_ref[...])


def _heads(h3, batch2d, wf, bf, wd1, bd1, wd2, bd2, wde, bde, wdg, bdg):
    return pl.pallas_call(
        _heads_body,
        out_shape=[
            jax.ShapeDtypeStruct((G, 1), _f32),
            jax.ShapeDtypeStruct((G, DOS_OUT), _f32),
            jax.ShapeDtypeStruct((G, DESC), _f32),
            jax.ShapeDtypeStruct((G, 1), _f32),
        ],
    )(h3, batch2d, wf, bf, wd1, bd1, wd2, bd2, wde, bde, wdg, bdg)


DESC = 8


_NBUF = 5


def _make_sc_gather_body(ne):
    ew = ne // _NW
    nch = ew // _B
    assert nch % _NBUF == 0

    def body(p_hbm, q_hbm, dst_hbm, src_hbm, z_hbm, idxd, idxs, pd, qs,
             *sems):
        c = lax.axis_index("c")
        s = lax.axis_index("s")
        wid = s * _NC + c
        base0 = wid * ew
        semi = sems[0:_NBUF]
        semg = sems[_NBUF:2 * _NBUF]
        semst = sems[2 * _NBUF:3 * _NBUF]

        def start_idx(j, b):
            base = base0 + j * _B
            pltpu.async_copy(dst_hbm.at[pl.ds(base, _B)], idxd.at[b], semi[b])
            pltpu.async_copy(src_hbm.at[pl.ds(base, _B)], idxs.at[b], semi[b])

        def wait_idx(b):
            pltpu.make_async_copy(dst_hbm.at[pl.ds(0, _B)], idxd.at[b],
                                  semi[b]).wait()
            pltpu.make_async_copy(src_hbm.at[pl.ds(0, _B)], idxs.at[b],
                                  semi[b]).wait()

        def start_gather(b):
            pltpu.async_copy(p_hbm.at[idxd.at[b]], pd.at[b], semg[b])
            pltpu.async_copy(q_hbm.at[idxs.at[b]], qs.at[b], semg[b])

        def wait_gather(b):
            pltpu.make_async_copy(p_hbm.at[idxd.at[b]], pd.at[b],
                                  semg[b]).wait()
            pltpu.make_async_copy(q_hbm.at[idxs.at[b]], qs.at[b],
                                  semg[b]).wait()

        def start_store(j, b):
            base = base0 + j * _B
            pltpu.async_copy(pd.at[b], z_hbm.at[pl.ds(base, _B)], semst[b])

        def wait_store(b):
            pltpu.make_async_copy(pd.at[b], z_hbm.at[pl.ds(0, _B)],
                                  semst[b]).wait()

        start_idx(0, 0)
        wait_idx(0)
        start_gather(0)
        start_idx(1, 1)

        def sub(j, b):
            bn = (b + 1) % _NBUF

            @pl.when(j + 1 < nch)
            def _():
                wait_idx(bn)

                @pl.when(j + 1 >= _NBUF)
                def _():
                    wait_store(bn)

                start_gather(bn)

            wait_gather(b)

            @pl.when(j + 2 < nch)
            def _():
                start_idx(j + 2, (b + 2) % _NBUF)

            pdb, qsb = pd.at[b], qs.at[b]

            def row(i, cc):
                for k in range(2 * D // 16):
                    sl = pl.ds(k * 16, 16)
                    pdb[i, sl] = pdb[i, sl] + qsb[i, sl]
                return cc

            lax.fori_loop(0, _B, row, 0)
            start_store(j, b)

        def bodyloop(i, cc):
            for t in range(_NBUF):
                sub(_NBUF * i + t, t)
            return cc

        lax.fori_loop(0, nch // _NBUF, bodyloop, 0)
        for b in range(_NBUF):
            wait_store(b)

    return body


@functools.lru_cache(maxsize=None)
def _make_sc_gather(ne):
    mesh = plsc.VectorSubcoreMesh(core_axis_name="c", subcore_axis_name="s",
                                  num_cores=_NC, num_subcores=_NS)
    return pl.kernel(
        _make_sc_gather_body(ne),
        out_type=jax.ShapeDtypeStruct((ne, 2 * D), _f32),
        mesh=mesh,
        scratch_types=[
            pltpu.VMEM((_NBUF, _B), jnp.int32),
            pltpu.VMEM((_NBUF, _B), jnp.int32),
            pltpu.VMEM((_NBUF, _B, 2 * D), _f32),
            pltpu.VMEM((_NBUF, _B, 2 * D), _f32),
        ] + [pltpu.SemaphoreType.DMA] * (3 * _NBUF),
    )


def _sc_gather(p, q, dst, src):
    return _make_sc_gather(dst.shape[0])(p, q, dst, src)


def _make_sc_scatter_body(ne):
    ew = ne // _NW
    nch = ew // _B
    assert nch % _NBUF == 0

    def body(msg_hbm, dst_hbm, out_hbm, acc, zbuf, idx, mv, *sems):
        c = lax.axis_index("c")
        s = lax.axis_index("s")
        seml = sems[0:_NBUF]
        semsc = sems[_NBUF:2 * _NBUF]

        def zrow(i, cc):
            for k in range(D // 16):
                zbuf[i, pl.ds(k * 16, 16)] = jnp.zeros((16,), _f32)
            return cc

        lax.fori_loop(0, _ZB, zrow, 0)
        for j in range(_RPT // _ZB):
            pltpu.sync_copy(zbuf, acc.at[pl.ds(s * _RPT + j * _ZB, _ZB)])
        plsc.subcore_barrier()

        base0 = (c * _NS + s) * ew

        def start_load(j, b):
            base = base0 + j * _B
            pltpu.async_copy(dst_hbm.at[pl.ds(base, _B)], idx.at[b], seml[b])
            pltpu.async_copy(msg_hbm.at[pl.ds(base, _B)], mv.at[b], seml[b])

        def wait_load(b):
            pltpu.make_async_copy(dst_hbm.at[pl.ds(0, _B)], idx.at[b],
                                  seml[b]).wait()
            pltpu.make_async_copy(msg_hbm.at[pl.ds(0, _B)], mv.at[b],
                                  seml[b]).wait()

        def start_scat(b):
            pltpu.async_copy(mv.at[b], acc.at[idx.at[b]], semsc[b], add=True)

        def wait_scat(b):
            pltpu.make_async_copy(mv.at[b], acc.at[idx.at[b]],
                                  semsc[b]).wait()

        start_load(0, 0)

        def sub(j, b):
            bn = (b + 1) % _NBUF

            @pl.when(j + 1 < nch)
            def _():
                @pl.when(j + 1 >= _NBUF)
                def _():
                    wait_scat(bn)
                start_load(j + 1, bn)

            wait_load(b)
            start_scat(b)

        def bodyloop(i, cc):
            for t in range(_NBUF):
                sub(_NBUF * i + t, t)
            return cc

        lax.fori_loop(0, nch // _NBUF, bodyloop, 0)
        for b in range(_NBUF):
            wait_scat(b)
        plsc.subcore_barrier()

        for j in range(_RPT // _ZB):
            r0 = s * _RPT + j * _ZB
            pltpu.sync_copy(acc.at[pl.ds(r0, _ZB)],
                            out_hbm.at[c, pl.ds(r0, _ZB)])

    return body


@functools.lru_cache(maxsize=None)
def _make_sc_scatter(ne):
    mesh = plsc.VectorSubcoreMesh(core_axis_name="c", subcore_axis_name="s",
                                  num_cores=_NC, num_subcores=_NS)
    return pl.kernel(
        _make_sc_scatter_body(ne),
        out_type=jax.ShapeDtypeStruct((_NC, _NP, D), _f32),
        mesh=mesh,
        scratch_types=[
            pltpu.VMEM_SHARED((_NP, D), _f32),
            pltpu.VMEM((_ZB, D), _f32),
            pltpu.VMEM((_NBUF, _B), jnp.int32),
            pltpu.VMEM((_NBUF, _B, D), _f32),
        ] + [pltpu.SemaphoreType.DMA] * (2 * _NBUF),
    )


def _sc_scatter(msg, dst):
    return _make_sc_scatter(dst.shape[0])(msg, dst)


def kernel(x, edge_index, edge_attr, batch, params):
    src = edge_index[0]
    dst = edge_index[1]
    convs = params['convs']

    wd = [jnp.concatenate([cv['Wg'][:D], cv['Wc'][:D]], axis=1)
          for cv in convs]
    ws = [jnp.concatenate([cv['Wg'][D:2 * D], cv['Wc'][D:2 * D]], axis=1)
          for cv in convs]
    wge = [cv['Wg'][2 * D:] for cv in convs]
    wce = [cv['Wc'][2 * D:] for cv in convs]
    bg = [cv['bg'].reshape(1, D) for cv in convs]
    bc = [cv['bc'].reshape(1, D) for cv in convs]

    h, p, q = _embed_proj(x, params['W_embed'],
                          params['b_embed'].reshape(1, D), wd[0], ws[0])

    eh = E // 2
    dsta, dstb = dst[:eh], dst[eh:]
    srca, srcb = src[:eh], src[eh:]

    for l in range(3):
        za = _sc_gather(p, q, dsta, srca)
        zb = _sc_gather(p, q, dstb, srcb)
        ma = _msg(za, edge_attr, wge[l], wce[l], bg[l], bc[l], half=0)
        mb = _msg(zb, edge_attr, wge[l], wce[l], bg[l], bc[l], half=1)
        pa = _sc_scatter(ma, dsta)
        pb = _sc_scatter(mb, dstb)
        if l < 2:
            h, p, q = _update_proj(h, pa, pb, wd[l + 1], ws[l + 1])
        else:
            h3, site = _finalh(h, pa, pb, params['W_site'],
                               params['b_site'].reshape(1, 1))

    formation, dos, desc, dgh = _heads(
        h3, batch.reshape(1, N),
        params['W_form'], params['b_form'].reshape(1, 1),
        params['W_dos1'], params['b_dos1'].reshape(1, DOS_H),
        params['W_dos2'], params['b_dos2'].reshape(1, DOS_OUT),
        params['W_desc'], params['b_desc'].reshape(1, DESC),
        params['W_dgh'], params['b_dgh'].reshape(1, 1),
    )
    return (formation, site, dos, desc, dgh)
